# Initial kernel scaffold; baseline (speedup 1.0000x reference)
import jax
import jax.numpy as jnp
from jax.experimental import pallas as pl


def kernel(points_image, grid, depth_map):
    raise NotImplementedError("write your pallas kernel here")


# speedup vs baseline: 1.5001x; 1.5001x over previous
import jax
import jax.numpy as jnp
from jax.experimental import pallas as pl
from jax.experimental.pallas import tpu as pltpu

_N = 131072
_G = 1024
_BNL = 512
_EPS = 1e-6
_INV_LN2 = 1.4426950408889634


def _rbf_kernel(pts_ref, gx_ref, gy_ref, dep_ref, o_ref):
    px = pts_ref[0:1, :]
    py = pts_ref[1:2, :]
    dx = gx_ref[...] - px + _EPS
    dy = gy_ref[...] - py + _EPS
    d = jnp.sqrt(dx * dx + dy * dy)
    e = jnp.exp2(d * (-_INV_LN2 / 1.0))
    den = jnp.sum(e, axis=0, keepdims=True)
    num = jnp.sum(e * dep_ref[...], axis=0, keepdims=True)
    o_ref[...] = (num / den).reshape(1, 1, _BNL)


def kernel(points_image, grid, depth_map):
    n = points_image.shape[0]
    g = grid.shape[0]
    pts_t = points_image.T
    gxb = jnp.broadcast_to(grid[:, 0:1], (g, _BNL))
    gyb = jnp.broadcast_to(grid[:, 1:2], (g, _BNL))
    depb = jnp.broadcast_to(depth_map.reshape(g, 1), (g, _BNL))
    steps = n // _BNL
    out = pl.pallas_call(
        _rbf_kernel,
        out_shape=jax.ShapeDtypeStruct((steps, 1, _BNL), jnp.float32),
        grid=(steps,),
        in_specs=[
            pl.BlockSpec((2, _BNL), lambda i: (0, i)),
            pl.BlockSpec((g, _BNL), lambda i: (0, 0)),
            pl.BlockSpec((g, _BNL), lambda i: (0, 0)),
            pl.BlockSpec((g, _BNL), lambda i: (0, 0)),
        ],
        out_specs=pl.BlockSpec((1, 1, _BNL), lambda i: (i, 0, 0)),
        compiler_params=pltpu.CompilerParams(
            dimension_semantics=("parallel",),
        ),
        name="grid_rbf",
    )(pts_t, gxb, gyb, depb)
    return out.reshape(n, 1)


# speedup vs baseline: 2.8295x; 1.8862x over previous
import jax
import jax.numpy as jnp
from jax.experimental import pallas as pl
from jax.experimental.pallas import tpu as pltpu

_G = 1024
_BNL = 512
_GC = 128
_EPS = 1e-6
_INV_LN2 = 1.4426950408889634
_TINY = 1e-35


def _rbf_kernel(pts_ref, gx_ref, gy_ref, dep_ref, o_ref):
    pxe = pts_ref[0:1, :] - _EPS
    pye = pts_ref[1:2, :] - _EPS
    den_acc = jnp.zeros((8, _BNL), jnp.float32)
    num_acc = jnp.zeros((8, _BNL), jnp.float32)
    for c in range(_G // _GC):
        gx = gx_ref[c * _GC:(c + 1) * _GC, :]
        gy = gy_ref[c * _GC:(c + 1) * _GC, :]
        dep = dep_ref[c * _GC:(c + 1) * _GC, :]
        dx = gx - pxe
        dy = gy - pye
        s = dx * dx + dy * dy
        d = s * jax.lax.rsqrt(s + _TINY)
        e = jnp.exp2(d * (-_INV_LN2))
        den_acc = den_acc + jnp.sum(e.reshape(-1, 8, _BNL), axis=0)
        num_acc = num_acc + jnp.sum((e * dep).reshape(-1, 8, _BNL), axis=0)
    den = jnp.sum(den_acc, axis=0, keepdims=True)
    num = jnp.sum(num_acc, axis=0, keepdims=True)
    o_ref[...] = (num / den).reshape(1, 1, _BNL)


def kernel(points_image, grid, depth_map):
    n = points_image.shape[0]
    g = grid.shape[0]
    pts_t = points_image.T
    gxb = jnp.broadcast_to(grid[:, 0:1], (g, _BNL))
    gyb = jnp.broadcast_to(grid[:, 1:2], (g, _BNL))
    depb = jnp.broadcast_to(depth_map.reshape(g, 1), (g, _BNL))
    steps = n // _BNL
    half = steps // 2
    out = pl.pallas_call(
        _rbf_kernel,
        out_shape=jax.ShapeDtypeStruct((steps, 1, _BNL), jnp.float32),
        grid=(2, half),
        in_specs=[
            pl.BlockSpec((2, _BNL), lambda i, j: (0, i * half + j)),
            pl.BlockSpec((g, _BNL), lambda i, j: (0, 0)),
            pl.BlockSpec((g, _BNL), lambda i, j: (0, 0)),
            pl.BlockSpec((g, _BNL), lambda i, j: (0, 0)),
        ],
        out_specs=pl.BlockSpec((1, 1, _BNL), lambda i, j: (i * half + j, 0, 0)),
        compiler_params=pltpu.CompilerParams(
            dimension_semantics=("parallel", "arbitrary"),
        ),
        name="grid_rbf",
    )(pts_t, gxb, gyb, depb)
    return out.reshape(n, 1)


# speedup vs baseline: 4.2172x; 1.4904x over previous
import jax
import jax.numpy as jnp
from jax.experimental import pallas as pl
from jax.experimental.pallas import tpu as pltpu

_G = 1024
_BNL = 512
_GC = 128
_C = 1.4426950408889634


def _rbf_kernel(pts_ref, gx_ref, gy_ref, w_ref, o_ref, gxs_ref, gys_ref):
    @pl.when(pl.program_id(0) == 0)
    def _init():
        gxs_ref[...] = (gx_ref[...] * _C).astype(jnp.bfloat16)
        gys_ref[...] = (gy_ref[...] * _C).astype(jnp.bfloat16)

    px = (pts_ref[0:1, :] * _C).astype(jnp.bfloat16)
    py = (pts_ref[1:2, :] * _C).astype(jnp.bfloat16)
    acc = jnp.zeros((2, _BNL), jnp.float32)
    for c in range(_G // _GC):
        sl = slice(c * _GC, (c + 1) * _GC)
        dx = gxs_ref[sl, :] - px
        dy = gys_ref[sl, :] - py
        s = jnp.maximum(dx * dx + dy * dy, jnp.bfloat16(1e-12))
        arg = (-s) * jax.lax.rsqrt(s)
        e = jnp.exp2(arg)
        acc = acc + jnp.dot(w_ref[:, sl], e, preferred_element_type=jnp.float32)
    o_ref[...] = (acc[0:1, :] / acc[1:2, :]).reshape(1, 1, _BNL)


def kernel(points_image, grid, depth_map):
    n = points_image.shape[0]
    g = grid.shape[0]
    pts_t = points_image.T
    gxb = jnp.broadcast_to(grid[:, 0:1], (g, _BNL))
    gyb = jnp.broadcast_to(grid[:, 1:2], (g, _BNL))
    w = jnp.stack([depth_map.reshape(-1), jnp.ones((g,), jnp.float32)]).astype(jnp.bfloat16)
    steps = n // _BNL
    out = pl.pallas_call(
        _rbf_kernel,
        out_shape=jax.ShapeDtypeStruct((steps, 1, _BNL), jnp.float32),
        grid=(steps,),
        in_specs=[
            pl.BlockSpec((2, _BNL), lambda i: (0, i)),
            pl.BlockSpec((g, _BNL), lambda i: (0, 0)),
            pl.BlockSpec((g, _BNL), lambda i: (0, 0)),
            pl.BlockSpec((2, g), lambda i: (0, 0)),
        ],
        out_specs=pl.BlockSpec((1, 1, _BNL), lambda i: (i, 0, 0)),
        scratch_shapes=[
            pltpu.VMEM((g, _BNL), jnp.bfloat16),
            pltpu.VMEM((g, _BNL), jnp.bfloat16),
        ],
        compiler_params=pltpu.CompilerParams(
            dimension_semantics=("arbitrary",),
        ),
        name="grid_rbf",
    )(pts_t, gxb, gyb, w)
    return out.reshape(n, 1)


# speedup vs baseline: 5.1633x; 1.2243x over previous
import jax
import jax.numpy as jnp
from jax.experimental import pallas as pl
from jax.experimental.pallas import tpu as pltpu

_G = 1024
_BW = 512
_NH = 4
_BNL = _BW * _NH
_GC = 128
_C = 1.4426950408889634


def _prep_kernel(gx_ref, gy_ref, gxs_ref, gys_ref):
    gxs_ref[...] = (gx_ref[...] * _C).astype(jnp.bfloat16)
    gys_ref[...] = (gy_ref[...] * _C).astype(jnp.bfloat16)


def _rbf_kernel(pts_ref, gxs_ref, gys_ref, w_ref, o_ref):
    for h in range(_NH):
        lo = h * _BW
        px = (pts_ref[0:1, lo:lo + _BW] * _C).astype(jnp.bfloat16)
        py = (pts_ref[1:2, lo:lo + _BW] * _C).astype(jnp.bfloat16)
        acc = jnp.zeros((2, _BW), jnp.float32)
        for c in range(_G // _GC):
            sl = slice(c * _GC, (c + 1) * _GC)
            dx = gxs_ref[sl, :] - px
            dy = gys_ref[sl, :] - py
            s = jnp.maximum(dx * dx + dy * dy, jnp.bfloat16(1e-12))
            arg = (-s) * jax.lax.rsqrt(s)
            e = jnp.exp2(arg)
            acc = acc + jnp.dot(w_ref[:, sl], e, preferred_element_type=jnp.float32)
        o_ref[0, 0, lo:lo + _BW] = (acc[0:1, :] / acc[1:2, :]).reshape(_BW)


def kernel(points_image, grid, depth_map):
    n = points_image.shape[0]
    g = grid.shape[0]
    pts_t = points_image.T
    gxb = jnp.broadcast_to(grid[:, 0:1], (g, _BW))
    gyb = jnp.broadcast_to(grid[:, 1:2], (g, _BW))
    w = jnp.stack([depth_map.reshape(-1), jnp.ones((g,), jnp.float32)]).astype(jnp.bfloat16)
    gxs, gys = pl.pallas_call(
        _prep_kernel,
        out_shape=(
            jax.ShapeDtypeStruct((g, _BW), jnp.bfloat16),
            jax.ShapeDtypeStruct((g, _BW), jnp.bfloat16),
        ),
        name="grid_rbf_prep",
    )(gxb, gyb)
    steps = n // _BNL
    out = pl.pallas_call(
        _rbf_kernel,
        out_shape=jax.ShapeDtypeStruct((steps, 1, _BNL), jnp.float32),
        grid=(steps,),
        in_specs=[
            pl.BlockSpec((2, _BNL), lambda i: (0, i)),
            pl.BlockSpec((g, _BW), lambda i: (0, 0)),
            pl.BlockSpec((g, _BW), lambda i: (0, 0)),
            pl.BlockSpec((2, g), lambda i: (0, 0)),
        ],
        out_specs=pl.BlockSpec((1, 1, _BNL), lambda i: (i, 0, 0)),
        compiler_params=pltpu.CompilerParams(
            dimension_semantics=("arbitrary",),
        ),
        name="grid_rbf",
    )(pts_t, gxs, gys, w)
    return out.reshape(n, 1)


# speedup vs baseline: 5.3583x; 1.0378x over previous
import jax
import jax.numpy as jnp
from jax.experimental import pallas as pl
from jax.experimental.pallas import tpu as pltpu

_G = 1024
_BW = 512
_NH = 8
_BNL = _BW * _NH
_GC = 256
_C = 1.4426950408889634


def _prep_kernel(gx_ref, gy_ref, gxs_ref, gys_ref):
    gxs_ref[...] = (gx_ref[...] * _C).astype(jnp.bfloat16)
    gys_ref[...] = (gy_ref[...] * _C).astype(jnp.bfloat16)


def _rbf_kernel(pts_ref, gxs_ref, gys_ref, w_ref, o_ref):
    for h in range(_NH):
        lo = h * _BW
        px = (pts_ref[0:1, lo:lo + _BW] * _C).astype(jnp.bfloat16)
        py = (pts_ref[1:2, lo:lo + _BW] * _C).astype(jnp.bfloat16)
        acc = jnp.zeros((2, _BW), jnp.float32)
        for c in range(_G // _GC):
            sl = slice(c * _GC, (c + 1) * _GC)
            dx = gxs_ref[sl, :] - px
            dy = gys_ref[sl, :] - py
            s = jnp.maximum(dx * dx + dy * dy, jnp.bfloat16(1e-12))
            arg = (-s) * jax.lax.rsqrt(s)
            e = jnp.exp2(arg)
            acc = acc + jnp.dot(w_ref[:, sl], e, preferred_element_type=jnp.float32)
        o_ref[0, 0, lo:lo + _BW] = (acc[0:1, :] / acc[1:2, :]).reshape(_BW)


def kernel(points_image, grid, depth_map):
    n = points_image.shape[0]
    g = grid.shape[0]
    pts_t = points_image.T
    gxb = jnp.broadcast_to(grid[:, 0:1], (g, _BW))
    gyb = jnp.broadcast_to(grid[:, 1:2], (g, _BW))
    w = jnp.stack([depth_map.reshape(-1), jnp.ones((g,), jnp.float32)]).astype(jnp.bfloat16)
    gxs, gys = pl.pallas_call(
        _prep_kernel,
        out_shape=(
            jax.ShapeDtypeStruct((g, _BW), jnp.bfloat16),
            jax.ShapeDtypeStruct((g, _BW), jnp.bfloat16),
        ),
        name="grid_rbf_prep",
    )(gxb, gyb)
    steps = n // _BNL
    out = pl.pallas_call(
        _rbf_kernel,
        out_shape=jax.ShapeDtypeStruct((steps, 1, _BNL), jnp.float32),
        grid=(steps,),
        in_specs=[
            pl.BlockSpec((2, _BNL), lambda i: (0, i)),
            pl.BlockSpec((g, _BW), lambda i: (0, 0)),
            pl.BlockSpec((g, _BW), lambda i: (0, 0)),
            pl.BlockSpec((2, g), lambda i: (0, 0)),
        ],
        out_specs=pl.BlockSpec((1, 1, _BNL), lambda i: (i, 0, 0)),
        compiler_params=pltpu.CompilerParams(
            dimension_semantics=("arbitrary",),
        ),
        name="grid_rbf",
    )(pts_t, gxs, gys, w)
    return out.reshape(n, 1)


# speedup vs baseline: 5.6059x; 1.0462x over previous
import jax
import jax.numpy as jnp
from jax.experimental import pallas as pl
from jax.experimental.pallas import tpu as pltpu

_G = 1024
_BW = 512
_NH = 8
_BNL = _BW * _NH
_GC = 128
_POLY_FRAC = 8
_C = 1.4426950408889634
_E3 = -0.026609474993353886
_E2 = 0.20499048397158054
_E1 = -0.6779864814811415
_E0 = 0.9989289915182401


def _prep_kernel(gx_ref, gy_ref, gxs_ref, gys_ref):
    gxs_ref[...] = (gx_ref[...] * _C).astype(jnp.bfloat16)
    gys_ref[...] = (gy_ref[...] * _C).astype(jnp.bfloat16)


def _rbf_kernel(pts_ref, gxs_ref, gys_ref, w_ref, o_ref):
    for h in range(_NH):
        lo = h * _BW
        px = (pts_ref[0:1, lo:lo + _BW] * _C).astype(jnp.bfloat16)
        py = (pts_ref[1:2, lo:lo + _BW] * _C).astype(jnp.bfloat16)
        acc = jnp.zeros((2, _BW), jnp.float32)
        for c in range(_G // _GC):
            sl = slice(c * _GC, (c + 1) * _GC)
            dx = gxs_ref[sl, :] - px
            dy = gys_ref[sl, :] - py
            s = jnp.maximum(dx * dx + dy * dy, jnp.bfloat16(1e-12))
            r = jax.lax.rsqrt(s)
            if c % 8 < _POLY_FRAC:
                u = s * r
                e = ((jnp.bfloat16(_E3) * u + jnp.bfloat16(_E2)) * u
                     + jnp.bfloat16(_E1)) * u + jnp.bfloat16(_E0)
            else:
                e = jnp.exp2((-s) * r)
            acc = acc + jnp.dot(w_ref[:, sl], e, preferred_element_type=jnp.float32)
        o_ref[0, 0, lo:lo + _BW] = (acc[0:1, :] / acc[1:2, :]).reshape(_BW)


def kernel(points_image, grid, depth_map):
    n = points_image.shape[0]
    g = grid.shape[0]
    pts_t = points_image.T
    gxb = jnp.broadcast_to(grid[:, 0:1], (g, _BW))
    gyb = jnp.broadcast_to(grid[:, 1:2], (g, _BW))
    w = jnp.stack([depth_map.reshape(-1), jnp.ones((g,), jnp.float32)]).astype(jnp.bfloat16)
    gxs, gys = pl.pallas_call(
        _prep_kernel,
        out_shape=(
            jax.ShapeDtypeStruct((g, _BW), jnp.bfloat16),
            jax.ShapeDtypeStruct((g, _BW), jnp.bfloat16),
        ),
        name="grid_rbf_prep",
    )(gxb, gyb)
    steps = n // _BNL
    out = pl.pallas_call(
        _rbf_kernel,
        out_shape=jax.ShapeDtypeStruct((steps, 1, _BNL), jnp.float32),
        grid=(steps,),
        in_specs=[
            pl.BlockSpec((2, _BNL), lambda i: (0, i)),
            pl.BlockSpec((g, _BW), lambda i: (0, 0)),
            pl.BlockSpec((g, _BW), lambda i: (0, 0)),
            pl.BlockSpec((2, g), lambda i: (0, 0)),
        ],
        out_specs=pl.BlockSpec((1, 1, _BNL), lambda i: (i, 0, 0)),
        compiler_params=pltpu.CompilerParams(
            dimension_semantics=("arbitrary",),
        ),
        name="grid_rbf",
    )(pts_t, gxs, gys, w)
    return out.reshape(n, 1)


# speedup vs baseline: 5.6096x; 1.0007x over previous
import jax
import jax.numpy as jnp
from jax.experimental import pallas as pl
from jax.experimental.pallas import tpu as pltpu

_G = 1024
_BW = 512
_NH = 8
_BNL = _BW * _NH
_GC = 128
_C = 1.4426950408889634
_E3 = -0.026609474993353886
_E2 = 0.20499048397158054
_E1 = -0.6779864814811415
_E0 = 0.9989289915182401


def _prep_kernel(gx_ref, gy_ref, gxs_ref, gys_ref):
    gxs_ref[...] = (gx_ref[...] * _C).astype(jnp.bfloat16)
    gys_ref[...] = (gy_ref[...] * _C).astype(jnp.bfloat16)


def _rbf_kernel(pts_ref, gxs_ref, gys_ref, w_ref, o_ref):
    for h in range(_NH):
        lo = h * _BW
        px = (pts_ref[0:1, lo:lo + _BW] * _C).astype(jnp.bfloat16)
        py = (pts_ref[1:2, lo:lo + _BW] * _C).astype(jnp.bfloat16)
        acc = jnp.zeros((2, _BW), jnp.float32)
        for c in range(_G // _GC):
            sl = slice(c * _GC, (c + 1) * _GC)
            dx = gxs_ref[sl, :] - px
            dy = gys_ref[sl, :] - py
            s = jnp.maximum(dx * dx + dy * dy, jnp.bfloat16(1e-12))
            u = s * jax.lax.rsqrt(s)
            e = ((jnp.bfloat16(_E3) * u + jnp.bfloat16(_E2)) * u
                 + jnp.bfloat16(_E1)) * u + jnp.bfloat16(_E0)
            acc = acc + jnp.dot(w_ref[:, sl], e, preferred_element_type=jnp.float32)
        o_ref[0, 0, lo:lo + _BW] = (acc[0:1, :] / acc[1:2, :]).reshape(_BW)


def kernel(points_image, grid, depth_map):
    n = points_image.shape[0]
    g = grid.shape[0]
    pts_t = points_image.T
    gxb = jnp.broadcast_to(grid[:, 0:1], (g, _BW))
    gyb = jnp.broadcast_to(grid[:, 1:2], (g, _BW))
    w = jnp.stack([depth_map.reshape(-1), jnp.ones((g,), jnp.float32)]).astype(jnp.bfloat16)
    gxs, gys = pl.pallas_call(
        _prep_kernel,
        out_shape=(
            jax.ShapeDtypeStruct((g, _BW), jnp.bfloat16),
            jax.ShapeDtypeStruct((g, _BW), jnp.bfloat16),
        ),
        name="grid_rbf_prep",
    )(gxb, gyb)
    steps = n // _BNL
    out = pl.pallas_call(
        _rbf_kernel,
        out_shape=jax.ShapeDtypeStruct((steps, 1, _BNL), jnp.float32),
        grid=(steps,),
        in_specs=[
            pl.BlockSpec((2, _BNL), lambda i: (0, i)),
            pl.BlockSpec((g, _BW), lambda i: (0, 0)),
            pl.BlockSpec((g, _BW), lambda i: (0, 0)),
            pl.BlockSpec((2, g), lambda i: (0, 0)),
        ],
        out_specs=pl.BlockSpec((1, 1, _BNL), lambda i: (i, 0, 0)),
        compiler_params=pltpu.CompilerParams(
            dimension_semantics=("arbitrary",),
        ),
        name="grid_rbf",
    )(pts_t, gxs, gys, w)
    return out.reshape(n, 1)


# speedup vs baseline: 6.3429x; 1.1307x over previous
import jax
import jax.numpy as jnp
from jax.experimental import pallas as pl
from jax.experimental.pallas import tpu as pltpu

_G = 1024
_BW = 512
_NH = 8
_BNL = _BW * _NH
_GC = 128
_C = 1.4426950408889634
_E3 = -0.026609474993353886
_E2 = 0.20499048397158054
_E1 = -0.6779864814811415
_E0 = 0.9989289915182401
_Q2 = 0.1180305252514819
_Q1 = -0.6049375351617171
_Q0 = 0.9890284739868678


def _prep_kernel(gx_ref, gy_ref, gxs_ref, gys_ref):
    gxs_ref[...] = (gx_ref[...] * _C).astype(jnp.bfloat16)
    gys_ref[...] = (gy_ref[...] * _C).astype(jnp.bfloat16)


def _rbf_kernel(pts_ref, gxs_ref, gys_ref, w_ref, o_ref):
    for h in range(_NH):
        lo = h * _BW
        px = (pts_ref[0:1, lo:lo + _BW] * _C).astype(jnp.bfloat16)
        py = (pts_ref[1:2, lo:lo + _BW] * _C).astype(jnp.bfloat16)
        acc = jnp.zeros((2, _BW), jnp.float32)
        for c in range(_G // _GC):
            sl = slice(c * _GC, (c + 1) * _GC)
            dx = gxs_ref[sl, :] - px
            dy = gys_ref[sl, :] - py
            s = jnp.maximum(dx * dx + dy * dy, jnp.bfloat16(1e-12))
            u = s * jax.lax.rsqrt(s)
            e = (jnp.bfloat16(_Q2) * u + jnp.bfloat16(_Q1)) * u + jnp.bfloat16(_Q0)
            acc = acc + jnp.dot(w_ref[:, sl], e, preferred_element_type=jnp.float32)
        o_ref[0, 0, lo:lo + _BW] = (acc[0:1, :] / acc[1:2, :]).reshape(_BW)


def kernel(points_image, grid, depth_map):
    n = points_image.shape[0]
    g = grid.shape[0]
    pts_t = points_image.T
    gxb = jnp.broadcast_to(grid[:, 0:1], (g, _BW))
    gyb = jnp.broadcast_to(grid[:, 1:2], (g, _BW))
    w = jnp.stack([depth_map.reshape(-1), jnp.ones((g,), jnp.float32)]).astype(jnp.bfloat16)
    gxs, gys = pl.pallas_call(
        _prep_kernel,
        out_shape=(
            jax.ShapeDtypeStruct((g, _BW), jnp.bfloat16),
            jax.ShapeDtypeStruct((g, _BW), jnp.bfloat16),
        ),
        name="grid_rbf_prep",
    )(gxb, gyb)
    steps = n // _BNL
    out = pl.pallas_call(
        _rbf_kernel,
        out_shape=jax.ShapeDtypeStruct((steps, 1, _BNL), jnp.float32),
        grid=(steps,),
        in_specs=[
            pl.BlockSpec((2, _BNL), lambda i: (0, i)),
            pl.BlockSpec((g, _BW), lambda i: (0, 0)),
            pl.BlockSpec((g, _BW), lambda i: (0, 0)),
            pl.BlockSpec((2, g), lambda i: (0, 0)),
        ],
        out_specs=pl.BlockSpec((1, 1, _BNL), lambda i: (i, 0, 0)),
        compiler_params=pltpu.CompilerParams(
            dimension_semantics=("arbitrary",),
        ),
        name="grid_rbf",
    )(pts_t, gxs, gys, w)
    return out.reshape(n, 1)


# speedup vs baseline: 6.3452x; 1.0004x over previous
import jax
import jax.numpy as jnp
from jax.experimental import pallas as pl
from jax.experimental.pallas import tpu as pltpu

_G = 1024
_BW = 512
_NH = 8
_BNL = _BW * _NH
_GC = 128
_C = 1.4426950408889634
_Q2 = 0.1180305252514819
_Q1 = -0.6049375351617171
_Q0 = 0.9890284739868678


def _prep_kernel(gx_ref, gy_ref, gxs_ref, gys_ref):
    gxs_ref[...] = (gx_ref[...] * _C).astype(jnp.bfloat16)
    gys_ref[...] = (gy_ref[...] * _C).astype(jnp.bfloat16)


def _rbf_kernel(pts_ref, gxs_ref, gys_ref, w_ref, o_ref):
    for h in range(_NH):
        lo = h * _BW
        px = (pts_ref[0:1, lo:lo + _BW] * _C).astype(jnp.bfloat16)
        py = (pts_ref[1:2, lo:lo + _BW] * _C).astype(jnp.bfloat16)
        acc = jnp.zeros((2, _BW), jnp.float32)
        for c in range(_G // _GC):
            sl = slice(c * _GC, (c + 1) * _GC)
            dx = gxs_ref[sl, :] - px
            dy = gys_ref[sl, :] - py
            s = jnp.maximum(dx * dx + dy * dy, jnp.bfloat16(1e-12))
            u = s * jax.lax.rsqrt(s)
            e = (jnp.bfloat16(_Q2) * u + jnp.bfloat16(_Q1)) * u + jnp.bfloat16(_Q0)
            acc = acc + jnp.dot(w_ref[:, sl], e, preferred_element_type=jnp.float32)
        o_ref[0, 0, lo:lo + _BW] = (acc[0:1, :] / acc[1:2, :]).reshape(_BW)


def kernel(points_image, grid, depth_map):
    n = points_image.shape[0]
    g = grid.shape[0]
    pts_t = points_image.T
    gxb = jnp.broadcast_to(grid[:, 0:1], (g, _BW))
    gyb = jnp.broadcast_to(grid[:, 1:2], (g, _BW))
    w = jnp.stack([depth_map.reshape(-1), jnp.ones((g,), jnp.float32)]).astype(jnp.bfloat16)
    gxs, gys = pl.pallas_call(
        _prep_kernel,
        out_shape=(
            jax.ShapeDtypeStruct((g, _BW), jnp.bfloat16),
            jax.ShapeDtypeStruct((g, _BW), jnp.bfloat16),
        ),
        name="grid_rbf_prep",
    )(gxb, gyb)
    steps = n // _BNL
    out = pl.pallas_call(
        _rbf_kernel,
        out_shape=jax.ShapeDtypeStruct((steps, 1, _BNL), jnp.float32),
        grid=(steps,),
        in_specs=[
            pl.BlockSpec((2, _BNL), lambda i: (0, i)),
            pl.BlockSpec((g, _BW), lambda i: (0, 0)),
            pl.BlockSpec((g, _BW), lambda i: (0, 0)),
            pl.BlockSpec((2, g), lambda i: (0, 0)),
        ],
        out_specs=pl.BlockSpec((1, 1, _BNL), lambda i: (i, 0, 0)),
        compiler_params=pltpu.CompilerParams(
            dimension_semantics=("arbitrary",),
        ),
        name="grid_rbf",
    )(pts_t, gxs, gys, w)
    return out.reshape(n, 1)
